# baseline (speedup 1.0000x reference)
import jax
import jax.numpy as jnp
from jax.experimental import pallas as pl

N = 10000
D = 128
F = 8
H = 64
K = 16
BLK = 5000


def _fused_kernel(x_ref, w1_ref, b1_ref, w2s_ref, b2e_ref, consts_ref,
                  wfull_ref, outb_ref, out_ref):
    xb = x_ref[...]
    h = jnp.maximum(
        jnp.dot(xb, w1_ref[...], preferred_element_type=jnp.float32)
        + b1_ref[...], 0.0)
    vE = (jnp.dot(h, w2s_ref[...], preferred_element_type=jnp.float32)
          + b2e_ref[...])

    t_tri = consts_ref[0:1, :]
    c0 = consts_ref[1:2, :]
    c1 = consts_ref[2:3, :]
    ls = consts_ref[3:4, :]
    bl = consts_ref[4:5, :]
    ch0 = consts_ref[5:6, :]
    ch1 = consts_ref[6:7, :]
    r = consts_ref[7:8, :]

    tri = jnp.maximum(vE - jnp.abs(vE - t_tri), 0.0)
    g0 = vE - c0
    g1 = vE - c1
    gau = jnp.exp((g0 * g0 + g1 * g1) * -0.5)
    lin = vE * ls + bl
    d = jnp.abs(vE - ch0) + jnp.abs(vE - ch1)
    hat = 1.0 / (1.0 + d) - 1.0 / (1.0 + jnp.abs(r - d))

    cc = jnp.concatenate([xb, tri, gau, lin, hat], axis=1)
    out = (jnp.dot(cc, wfull_ref[...], preferred_element_type=jnp.float32)
           + outb_ref[...])
    out_ref[...] = jnp.maximum(out, 0.0)


def kernel(x, edge_index, batch_idx, edge_slices, W1, b1, W2, b2, t_tri,
           c_gauss, w_line, b_line, c_hat, r_hat, out_W, out_b):
    FK = F * K

    S = (jnp.arange(FK, dtype=jnp.int32)[None, :] // K
         == jnp.arange(F, dtype=jnp.int32)[:, None]).astype(jnp.float32)
    W2S = W2 @ S
    b2E = jnp.repeat(b2, K)[None, :]

    r = jnp.abs(r_hat[0])
    consts = jnp.stack([
        jnp.tile(t_tri, F),
        jnp.tile(c_gauss[:, 0], F),
        jnp.tile(c_gauss[:, 1], F),
        jnp.tile(w_line[0] + w_line[1], F),
        jnp.tile(b_line, F),
        jnp.tile(c_hat[:, 0], F),
        jnp.tile(c_hat[:, 1], F),
        jnp.full((FK,), r, dtype=jnp.float32),
    ])

    j = jnp.arange(4 * FK, dtype=jnp.int32)
    g = j // FK
    rem = j % FK
    f = rem // K
    k = rem % K
    perm = f * (4 * K) + g * K + k
    Wfull = jnp.concatenate([out_W[:D], out_W[D:][perm]], axis=0)

    grid = (N // BLK,)
    return pl.pallas_call(
        _fused_kernel,
        grid=grid,
        in_specs=[
            pl.BlockSpec((BLK, D), lambda i: (i, 0)),
            pl.BlockSpec((D, H), lambda i: (0, 0)),
            pl.BlockSpec((1, H), lambda i: (0, 0)),
            pl.BlockSpec((H, FK), lambda i: (0, 0)),
            pl.BlockSpec((1, FK), lambda i: (0, 0)),
            pl.BlockSpec((8, FK), lambda i: (0, 0)),
            pl.BlockSpec((D + 4 * FK, D), lambda i: (0, 0)),
            pl.BlockSpec((1, D), lambda i: (0, 0)),
        ],
        out_specs=pl.BlockSpec((BLK, D), lambda i: (i, 0)),
        out_shape=jax.ShapeDtypeStruct((N, D), jnp.float32),
    )(x, W1, b1[None, :], W2S, b2E, consts, Wfull, out_b[None, :])


# speedup vs baseline: 1.0630x; 1.0630x over previous
import jax
import jax.numpy as jnp
from jax.experimental import pallas as pl

N = 10000
D = 128
F = 8
H = 64
K = 16
BLK = 2000


def _fused_kernel(x_ref, w1_ref, b1_ref, w2s_ref, b2e_ref, consts_ref,
                  wfull_ref, outb_ref, out_ref):
    xb = x_ref[...]
    h = jnp.maximum(
        jnp.dot(xb, w1_ref[...], preferred_element_type=jnp.float32)
        + b1_ref[...], 0.0)
    vE = (jnp.dot(h, w2s_ref[...], preferred_element_type=jnp.float32)
          + b2e_ref[...])

    t_tri = consts_ref[0:1, :]
    c0 = consts_ref[1:2, :]
    c1 = consts_ref[2:3, :]
    ls = consts_ref[3:4, :]
    bl = consts_ref[4:5, :]
    ch0 = consts_ref[5:6, :]
    ch1 = consts_ref[6:7, :]
    r = consts_ref[7:8, :]

    tri = jnp.maximum(vE - jnp.abs(vE - t_tri), 0.0)
    g0 = vE - c0
    g1 = vE - c1
    gau = jnp.exp((g0 * g0 + g1 * g1) * -0.5)
    lin = vE * ls + bl
    d = jnp.abs(vE - ch0) + jnp.abs(vE - ch1)
    hat = 1.0 / (1.0 + d) - 1.0 / (1.0 + jnp.abs(r - d))

    cc = jnp.concatenate([xb, tri, gau, lin, hat], axis=1)
    out = (jnp.dot(cc, wfull_ref[...], preferred_element_type=jnp.float32)
           + outb_ref[...])
    out_ref[...] = jnp.maximum(out, 0.0)


def kernel(x, edge_index, batch_idx, edge_slices, W1, b1, W2, b2, t_tri,
           c_gauss, w_line, b_line, c_hat, r_hat, out_W, out_b):
    FK = F * K

    S = (jnp.arange(FK, dtype=jnp.int32)[None, :] // K
         == jnp.arange(F, dtype=jnp.int32)[:, None]).astype(jnp.float32)
    W2S = W2 @ S
    b2E = jnp.repeat(b2, K)[None, :]

    r = jnp.abs(r_hat[0])
    consts = jnp.stack([
        jnp.tile(t_tri, F),
        jnp.tile(c_gauss[:, 0], F),
        jnp.tile(c_gauss[:, 1], F),
        jnp.tile(w_line[0] + w_line[1], F),
        jnp.tile(b_line, F),
        jnp.tile(c_hat[:, 0], F),
        jnp.tile(c_hat[:, 1], F),
        jnp.full((FK,), r, dtype=jnp.float32),
    ])

    j = jnp.arange(4 * FK, dtype=jnp.int32)
    g = j // FK
    rem = j % FK
    f = rem // K
    k = rem % K
    perm = f * (4 * K) + g * K + k
    Wfull = jnp.concatenate([out_W[:D], out_W[D:][perm]], axis=0)

    grid = (N // BLK,)
    return pl.pallas_call(
        _fused_kernel,
        grid=grid,
        in_specs=[
            pl.BlockSpec((BLK, D), lambda i: (i, 0)),
            pl.BlockSpec((D, H), lambda i: (0, 0)),
            pl.BlockSpec((1, H), lambda i: (0, 0)),
            pl.BlockSpec((H, FK), lambda i: (0, 0)),
            pl.BlockSpec((1, FK), lambda i: (0, 0)),
            pl.BlockSpec((8, FK), lambda i: (0, 0)),
            pl.BlockSpec((D + 4 * FK, D), lambda i: (0, 0)),
            pl.BlockSpec((1, D), lambda i: (0, 0)),
        ],
        out_specs=pl.BlockSpec((BLK, D), lambda i: (i, 0)),
        out_shape=jax.ShapeDtypeStruct((N, D), jnp.float32),
    )(x, W1, b1[None, :], W2S, b2E, consts, Wfull, out_b[None, :])
